# baseline (speedup 1.0000x reference)
import functools

import jax
import jax.numpy as jnp
from jax import lax
from jax.experimental import pallas as pl
from jax.experimental.pallas import tpu as pltpu
from jax.experimental.pallas import tpu_sc as plsc

_LANES = 16
_CHUNK = 128


def _node_tables(seq, fc_W, fc_b, w_top, w_bot, lin1_b):
    n, ft = seq.shape
    nh = w_top.shape[1]
    blk = 2000
    assert n % blk == 0

    def body(seq_ref, fcw_ref, fcb_ref, wt_ref, wb_ref, l1b_ref, a_ref, b_ref):
        z = jnp.dot(seq_ref[...], fcw_ref[...], preferred_element_type=jnp.float32)
        z = z + fcb_ref[...]
        a_ref[...] = jnp.dot(z, wt_ref[...], preferred_element_type=jnp.float32)
        b_ref[...] = (
            jnp.dot(z, wb_ref[...], preferred_element_type=jnp.float32) + l1b_ref[...]
        )

    full = lambda i: (0, 0)
    return pl.pallas_call(
        body,
        grid=(n // blk,),
        in_specs=[
            pl.BlockSpec((blk, ft), lambda i: (i, 0)),
            pl.BlockSpec((ft, nh), full),
            pl.BlockSpec((1, nh), full),
            pl.BlockSpec((nh, nh), full),
            pl.BlockSpec((nh, nh), full),
            pl.BlockSpec((1, nh), full),
        ],
        out_specs=[
            pl.BlockSpec((blk, nh), lambda i: (i, 0)),
            pl.BlockSpec((blk, nh), lambda i: (i, 0)),
        ],
        out_shape=[
            jax.ShapeDtypeStruct((n, nh), jnp.float32),
            jax.ShapeDtypeStruct((n, nh), jnp.float32),
        ],
    )(seq, fc_W, fc_b.reshape(1, nh), w_top, w_bot, lin1_b.reshape(1, nh))


def _edge_scores(a_tab, b_tab, row, col, w2, b2_vec, g_per_w):
    info = plsc.get_sparse_core_info()
    nc = info.num_cores
    e_pad = row.shape[0]
    nh = a_tab.shape[1]
    ngrp = _CHUNK // _LANES
    mesh = plsc.VectorSubcoreMesh(core_axis_name="c", subcore_axis_name="s")

    @functools.partial(
        pl.kernel,
        out_type=jax.ShapeDtypeStruct((e_pad,), jnp.float32),
        mesh=mesh,
        compiler_params=pltpu.CompilerParams(needs_layout_passes=False),
        scratch_types=[
            pltpu.VMEM((_CHUNK,), jnp.int32),
            pltpu.VMEM((_CHUNK,), jnp.int32),
            pltpu.VMEM((_CHUNK, nh), jnp.float32),
            pltpu.VMEM((_CHUNK, nh), jnp.float32),
            pltpu.VMEM((_CHUNK,), jnp.float32),
            pltpu.VMEM((nh,), jnp.float32),
            pltpu.VMEM((_LANES,), jnp.float32),
            pltpu.SemaphoreType.DMA,
            pltpu.SemaphoreType.DMA,
        ],
    )
    def k(a_hbm, b_hbm, row_hbm, col_hbm, w2_hbm, b2_hbm, out_hbm,
          idxr_v, idxc_v, a_v, b_v, o_v, w2_v, b2_v, sem_a, sem_b):
        wid = lax.axis_index("s") * nc + lax.axis_index("c")
        base = wid * (g_per_w * _CHUNK)
        pltpu.sync_copy(w2_hbm, w2_v)
        pltpu.sync_copy(b2_hbm, b2_v)
        lanes = lax.iota(jnp.int32, _LANES)

        def chunk(g, carry):
            off = base + g * _CHUNK
            pltpu.sync_copy(row_hbm.at[pl.ds(off, _CHUNK)], idxr_v)
            pltpu.sync_copy(col_hbm.at[pl.ds(off, _CHUNK)], idxc_v)
            cpa = pltpu.async_copy(a_hbm.at[idxr_v], a_v, sem_a)
            cpb = pltpu.async_copy(b_hbm.at[idxc_v], b_v, sem_b)
            cpa.wait()
            cpb.wait()

            def kstep(kb, accs):
                wv = w2_v[pl.ds(kb * _LANES, _LANES)]
                accs = list(accs)
                for j in range(_LANES):
                    colk = jnp.zeros((_LANES,), jnp.int32) + (kb * _LANES + j)
                    wk = wv[j]
                    for gi in range(ngrp):
                        rows = lanes + (gi * _LANES)
                        av = plsc.load_gather(a_v, [rows, colk])
                        bv = plsc.load_gather(b_v, [rows, colk])
                        h = jnp.maximum(av + bv, 0.0)
                        accs[gi] = accs[gi] + h * wk
                return tuple(accs)

            accs = lax.fori_loop(
                0, nh // _LANES, kstep,
                tuple(jnp.zeros((_LANES,), jnp.float32) for _ in range(ngrp)))
            b2 = b2_v[...]
            for gi in range(ngrp):
                o_v[pl.ds(gi * _LANES, _LANES)] = accs[gi] + b2
            pltpu.sync_copy(o_v, out_hbm.at[pl.ds(off, _CHUNK)])
            return carry

        lax.fori_loop(0, g_per_w, chunk, 0)

    return k(a_tab, b_tab, row, col, w2, b2_vec)


def kernel(seq, adj, fc_W, fc_b, lin1_W, lin1_b, lin2_W, lin2_b):
    e = adj.shape[1]
    nh = lin1_W.shape[0] // 2
    a_tab, b_tab = _node_tables(seq, fc_W, fc_b, lin1_W[:nh], lin1_W[nh:], lin1_b)

    info = plsc.get_sparse_core_info()
    nw = info.num_cores * info.num_subcores
    per = nw * _CHUNK
    g_per_w = -(-e // per)
    e_pad = g_per_w * per
    row = adj[0].astype(jnp.int32)
    col = adj[1].astype(jnp.int32)
    if e_pad != e:
        zpad = jnp.zeros((e_pad - e,), jnp.int32)
        row = jnp.concatenate([row, zpad])
        col = jnp.concatenate([col, zpad])

    w2 = lin2_W[:, 0]
    b2_vec = jnp.full((_LANES,), lin2_b[0], jnp.float32)
    out = _edge_scores(a_tab, b_tab, row, col, w2, b2_vec, g_per_w)
    return out[:e]


# speedup vs baseline: 1.1396x; 1.1396x over previous
import functools

import jax
import jax.numpy as jnp
from jax import lax
from jax.experimental import pallas as pl
from jax.experimental.pallas import tpu as pltpu
from jax.experimental.pallas import tpu_sc as plsc

_LANES = 16
_CHUNK = 128


def _node_tables(seq, fc_W, fc_b, w_top, w_bot, lin1_b):
    n, ft = seq.shape
    nh = w_top.shape[1]
    blk = 2000
    assert n % blk == 0

    def body(seq_ref, fcw_ref, fcb_ref, wt_ref, wb_ref, l1b_ref, a_ref, b_ref):
        z = jnp.dot(seq_ref[...], fcw_ref[...], preferred_element_type=jnp.float32)
        z = z + fcb_ref[...]
        a_ref[...] = jnp.dot(z, wt_ref[...], preferred_element_type=jnp.float32)
        b_ref[...] = (
            jnp.dot(z, wb_ref[...], preferred_element_type=jnp.float32) + l1b_ref[...]
        )

    full = lambda i: (0, 0)
    return pl.pallas_call(
        body,
        grid=(n // blk,),
        in_specs=[
            pl.BlockSpec((blk, ft), lambda i: (i, 0)),
            pl.BlockSpec((ft, nh), full),
            pl.BlockSpec((1, nh), full),
            pl.BlockSpec((nh, nh), full),
            pl.BlockSpec((nh, nh), full),
            pl.BlockSpec((1, nh), full),
        ],
        out_specs=[
            pl.BlockSpec((blk, nh), lambda i: (i, 0)),
            pl.BlockSpec((blk, nh), lambda i: (i, 0)),
        ],
        out_shape=[
            jax.ShapeDtypeStruct((n, nh), jnp.float32),
            jax.ShapeDtypeStruct((n, nh), jnp.float32),
        ],
    )(seq, fc_W, fc_b.reshape(1, nh), w_top, w_bot, lin1_b.reshape(1, nh))


def _edge_scores(a_tab, b_tab, row3, col3, w2s, b2_vec):
    info = plsc.get_sparse_core_info()
    nc = info.num_cores
    nw = nc * info.num_subcores
    g_per_w = row3.shape[1]
    nh = a_tab.shape[1]
    ngrp = _CHUNK // _LANES
    mesh = plsc.VectorSubcoreMesh(core_axis_name="c", subcore_axis_name="s")

    @functools.partial(
        pl.kernel,
        out_type=jax.ShapeDtypeStruct((nw, g_per_w, _CHUNK), jnp.float32),
        mesh=mesh,
        compiler_params=pltpu.CompilerParams(needs_layout_passes=False),
        scratch_types=[
            pltpu.VMEM((g_per_w, _CHUNK), jnp.int32),
            pltpu.VMEM((g_per_w, _CHUNK), jnp.int32),
            pltpu.VMEM((_CHUNK, nh), jnp.float32),
            pltpu.VMEM((_CHUNK, nh), jnp.float32),
            pltpu.VMEM((_CHUNK, nh), jnp.float32),
            pltpu.VMEM((_CHUNK, nh), jnp.float32),
            pltpu.VMEM((g_per_w, _CHUNK), jnp.float32),
            pltpu.VMEM((nh, _LANES), jnp.float32),
            pltpu.VMEM((_LANES,), jnp.float32),
            pltpu.SemaphoreType.DMA,
            pltpu.SemaphoreType.DMA,
            pltpu.SemaphoreType.DMA,
            pltpu.SemaphoreType.DMA,
        ],
    )
    def k(a_hbm, b_hbm, row_hbm, col_hbm, w2s_hbm, b2_hbm, out_hbm,
          idxr_v, idxc_v, a0_v, b0_v, a1_v, b1_v, o_v, w2s_v, b2_v,
          sa0, sb0, sa1, sb1):
        wid = lax.axis_index("s") * nc + lax.axis_index("c")
        pltpu.sync_copy(row_hbm.at[wid], idxr_v)
        pltpu.sync_copy(col_hbm.at[wid], idxc_v)
        pltpu.sync_copy(w2s_hbm, w2s_v)
        pltpu.sync_copy(b2_hbm, b2_v)
        lanes = lax.iota(jnp.int32, _LANES)
        b2 = b2_v[...]

        def init_g(g, carry):
            for gi in range(ngrp):
                o_v[g, pl.ds(gi * _LANES, _LANES)] = b2
            return carry

        lax.fori_loop(0, g_per_w, init_g, 0)

        def issue(g, a_buf, b_buf, sem_a, sem_b):
            pltpu.make_async_copy(a_hbm.at[idxr_v.at[g]], a_buf, sem_a).start()
            pltpu.make_async_copy(b_hbm.at[idxc_v.at[g]], b_buf, sem_b).start()

        def drain(g, a_buf, b_buf, sem_a, sem_b):
            pltpu.make_async_copy(a_hbm.at[idxr_v.at[g]], a_buf, sem_a).wait()
            pltpu.make_async_copy(b_hbm.at[idxc_v.at[g]], b_buf, sem_b).wait()

        def compute(g, a_buf, b_buf):
            def kstep(kk, carry):
                wk = w2s_v[kk]
                colk = jnp.zeros((_LANES,), jnp.int32) + kk
                for gi in range(ngrp):
                    rows = lanes + gi * _LANES
                    av = plsc.load_gather(a_buf, [rows, colk])
                    bv = plsc.load_gather(b_buf, [rows, colk])
                    h = jnp.maximum(av + bv, 0.0)
                    plsc.addupdate(o_v.at[g, pl.ds(gi * _LANES, _LANES)], h * wk)
                return carry

            lax.fori_loop(0, nh, kstep, 0)

        issue(0, a0_v, b0_v, sa0, sb0)

        def two_chunks(gg, carry):
            g0 = 2 * gg
            issue(g0 + 1, a1_v, b1_v, sa1, sb1)
            drain(g0, a0_v, b0_v, sa0, sb0)
            compute(g0, a0_v, b0_v)

            @pl.when(gg < g_per_w // 2 - 1)
            def _():
                issue(g0 + 2, a0_v, b0_v, sa0, sb0)

            drain(g0 + 1, a1_v, b1_v, sa1, sb1)
            compute(g0 + 1, a1_v, b1_v)
            return carry

        lax.fori_loop(0, g_per_w // 2, two_chunks, 0)
        pltpu.sync_copy(o_v, out_hbm.at[wid])

    return k(a_tab, b_tab, row3, col3, w2s, b2_vec)


def kernel(seq, adj, fc_W, fc_b, lin1_W, lin1_b, lin2_W, lin2_b):
    e = adj.shape[1]
    nh = lin1_W.shape[0] // 2
    a_tab, b_tab = _node_tables(seq, fc_W, fc_b, lin1_W[:nh], lin1_W[nh:], lin1_b)

    info = plsc.get_sparse_core_info()
    nw = info.num_cores * info.num_subcores
    per = nw * _CHUNK
    g_per_w = -(-e // per)
    g_per_w += g_per_w % 2
    e_pad = g_per_w * per
    row = adj[0].astype(jnp.int32)
    col = adj[1].astype(jnp.int32)
    if e_pad != e:
        zpad = jnp.zeros((e_pad - e,), jnp.int32)
        row = jnp.concatenate([row, zpad])
        col = jnp.concatenate([col, zpad])
    row3 = row.reshape(nw, g_per_w, _CHUNK)
    col3 = col.reshape(nw, g_per_w, _CHUNK)

    w2s = jnp.broadcast_to(lin2_W[:, :1], (lin2_W.shape[0], _LANES))
    b2_vec = jnp.full((_LANES,), lin2_b[0], jnp.float32)
    out = _edge_scores(a_tab, b_tab, row3, col3, w2s, b2_vec)
    return out.reshape(-1)[:e]


# speedup vs baseline: 3.7809x; 3.3177x over previous
import functools

import jax
import jax.numpy as jnp
from jax import lax
from jax.experimental import pallas as pl
from jax.experimental.pallas import tpu as pltpu
from jax.experimental.pallas import tpu_sc as plsc

_LANES = 16
_CHUNK = 128


def _node_tables(seq, fc_W, fc_b, w_top, w_bot, lin1_b):
    n, ft = seq.shape
    nh = w_top.shape[1]
    blk = 2000
    assert n % blk == 0

    def body(seq_ref, fcw_ref, fcb_ref, wt_ref, wb_ref, l1b_ref, a_ref, b_ref):
        z = jnp.dot(seq_ref[...], fcw_ref[...], preferred_element_type=jnp.float32)
        z = z + fcb_ref[...]
        a_ref[...] = jnp.dot(z, wt_ref[...], preferred_element_type=jnp.float32)
        b_ref[...] = (
            jnp.dot(z, wb_ref[...], preferred_element_type=jnp.float32) + l1b_ref[...]
        )

    full = lambda i: (0, 0)
    return pl.pallas_call(
        body,
        grid=(n // blk,),
        in_specs=[
            pl.BlockSpec((blk, ft), lambda i: (i, 0)),
            pl.BlockSpec((ft, nh), full),
            pl.BlockSpec((1, nh), full),
            pl.BlockSpec((nh, nh), full),
            pl.BlockSpec((nh, nh), full),
            pl.BlockSpec((1, nh), full),
        ],
        out_specs=[
            pl.BlockSpec((blk, nh), lambda i: (i, 0)),
            pl.BlockSpec((blk, nh), lambda i: (i, 0)),
        ],
        out_shape=[
            jax.ShapeDtypeStruct((n, nh), jnp.float32),
            jax.ShapeDtypeStruct((n, nh), jnp.float32),
        ],
    )(seq, fc_W, fc_b.reshape(1, nh), w_top, w_bot, lin1_b.reshape(1, nh))


def _edge_scores(a_tab, b_tab, row3, col3, w2s, b2_vec):
    info = plsc.get_sparse_core_info()
    nc = info.num_cores
    nw = nc * info.num_subcores
    g_per_w = row3.shape[1]
    nh = a_tab.shape[1]
    ngrp = _CHUNK // _LANES
    mesh = plsc.VectorSubcoreMesh(core_axis_name="c", subcore_axis_name="s")

    @functools.partial(
        pl.kernel,
        out_type=jax.ShapeDtypeStruct((nw, g_per_w, _CHUNK), jnp.float32),
        mesh=mesh,
        compiler_params=pltpu.CompilerParams(needs_layout_passes=False),
        scratch_types=[
            pltpu.VMEM((g_per_w, _CHUNK), jnp.int32),
            pltpu.VMEM((g_per_w, _CHUNK), jnp.int32),
            pltpu.VMEM((_CHUNK, nh), jnp.float32),
            pltpu.VMEM((_CHUNK, nh), jnp.float32),
            pltpu.VMEM((_CHUNK, nh), jnp.float32),
            pltpu.VMEM((_CHUNK, nh), jnp.float32),
            pltpu.VMEM((g_per_w, _CHUNK), jnp.float32),
            pltpu.VMEM((nh,), jnp.float32),
            pltpu.VMEM((_LANES,), jnp.float32),
            pltpu.SemaphoreType.DMA,
            pltpu.SemaphoreType.DMA,
            pltpu.SemaphoreType.DMA,
            pltpu.SemaphoreType.DMA,
        ],
    )
    def k(a_hbm, b_hbm, row_hbm, col_hbm, w2_hbm, b2_hbm, out_hbm,
          idxr_v, idxc_v, a0_v, b0_v, a1_v, b1_v, o_v, w2_v, b2_v,
          sa0, sb0, sa1, sb1):
        wid = lax.axis_index("s") * nc + lax.axis_index("c")
        pltpu.sync_copy(row_hbm.at[wid], idxr_v)
        pltpu.sync_copy(col_hbm.at[wid], idxc_v)
        pltpu.sync_copy(w2_hbm, w2_v)
        pltpu.sync_copy(b2_hbm, b2_v)
        lanes = lax.iota(jnp.int32, _LANES)
        b2 = b2_v[...]
        wseg = [w2_v[pl.ds(s * _LANES, _LANES)] for s in range(nh // _LANES)]
        last = jnp.full((_LANES,), _LANES - 1, jnp.int32)

        def issue(g, a_buf, b_buf, sem_a, sem_b):
            pltpu.make_async_copy(a_hbm.at[idxr_v.at[g]], a_buf, sem_a).start()
            pltpu.make_async_copy(b_hbm.at[idxc_v.at[g]], b_buf, sem_b).start()

        def drain(g, a_buf, b_buf, sem_a, sem_b):
            pltpu.make_async_copy(a_hbm.at[idxr_v.at[g]], a_buf, sem_a).wait()
            pltpu.make_async_copy(b_hbm.at[idxc_v.at[g]], b_buf, sem_b).wait()

        def compute(g, a_buf, b_buf):
            def egstep(eg, carry):
                r = jnp.zeros((_LANES,), jnp.float32)
                for j in range(_LANES):
                    e = eg * _LANES + j
                    acc = None
                    for s in range(nh // _LANES):
                        av = a_buf[e, pl.ds(s * _LANES, _LANES)]
                        bv = b_buf[e, pl.ds(s * _LANES, _LANES)]
                        h = jnp.maximum(av + bv, 0.0) * wseg[s]
                        acc = h if acc is None else acc + h
                    total = jnp.sum(acc)
                    r = jnp.where(lanes == j, total, r)
                o_v[g, pl.ds(eg * _LANES, _LANES)] = r + b2
                return carry

            lax.fori_loop(0, ngrp, egstep, 0)

        issue(0, a0_v, b0_v, sa0, sb0)

        def two_chunks(gg, carry):
            g0 = 2 * gg
            issue(g0 + 1, a1_v, b1_v, sa1, sb1)
            drain(g0, a0_v, b0_v, sa0, sb0)
            compute(g0, a0_v, b0_v)

            @pl.when(gg < g_per_w // 2 - 1)
            def _():
                issue(g0 + 2, a0_v, b0_v, sa0, sb0)

            drain(g0 + 1, a1_v, b1_v, sa1, sb1)
            compute(g0 + 1, a1_v, b1_v)
            return carry

        lax.fori_loop(0, g_per_w // 2, two_chunks, 0)
        pltpu.sync_copy(o_v, out_hbm.at[wid])

    return k(a_tab, b_tab, row3, col3, w2s, b2_vec)


def kernel(seq, adj, fc_W, fc_b, lin1_W, lin1_b, lin2_W, lin2_b):
    e = adj.shape[1]
    nh = lin1_W.shape[0] // 2
    a_tab, b_tab = _node_tables(seq, fc_W, fc_b, lin1_W[:nh], lin1_W[nh:], lin1_b)

    info = plsc.get_sparse_core_info()
    nw = info.num_cores * info.num_subcores
    per = nw * _CHUNK
    g_per_w = -(-e // per)
    g_per_w += g_per_w % 2
    e_pad = g_per_w * per
    row = adj[0].astype(jnp.int32)
    col = adj[1].astype(jnp.int32)
    if e_pad != e:
        zpad = jnp.zeros((e_pad - e,), jnp.int32)
        row = jnp.concatenate([row, zpad])
        col = jnp.concatenate([col, zpad])
    row3 = row.reshape(nw, g_per_w, _CHUNK)
    col3 = col.reshape(nw, g_per_w, _CHUNK)

    w2 = lin2_W[:, 0]
    b2_vec = jnp.full((_LANES,), lin2_b[0], jnp.float32)
    out = _edge_scores(a_tab, b_tab, row3, col3, w2, b2_vec)
    return out.reshape(-1)[:e]


# speedup vs baseline: 4.9650x; 1.3132x over previous
import functools

import jax
import jax.numpy as jnp
from jax import lax
from jax.experimental import pallas as pl
from jax.experimental.pallas import tpu as pltpu
from jax.experimental.pallas import tpu_sc as plsc

_LANES = 16
_CHUNK = 128
_DEPTH = 3


def _node_tables(seq, fc_W, fc_b, w_top, w_bot, lin1_b):
    n, ft = seq.shape
    nh = w_top.shape[1]
    blk = 2000
    assert n % blk == 0

    def body(seq_ref, fcw_ref, fcb_ref, wt_ref, wb_ref, l1b_ref, a_ref, b_ref):
        z = jnp.dot(seq_ref[...], fcw_ref[...], preferred_element_type=jnp.float32)
        z = z + fcb_ref[...]
        a_ref[...] = jnp.dot(
            z, wt_ref[...], preferred_element_type=jnp.float32
        ).astype(jnp.bfloat16)
        b_ref[...] = (
            jnp.dot(z, wb_ref[...], preferred_element_type=jnp.float32) + l1b_ref[...]
        ).astype(jnp.bfloat16)

    full = lambda i: (0, 0)
    return pl.pallas_call(
        body,
        grid=(n // blk,),
        in_specs=[
            pl.BlockSpec((blk, ft), lambda i: (i, 0)),
            pl.BlockSpec((ft, nh), full),
            pl.BlockSpec((1, nh), full),
            pl.BlockSpec((nh, nh), full),
            pl.BlockSpec((nh, nh), full),
            pl.BlockSpec((1, nh), full),
        ],
        out_specs=[
            pl.BlockSpec((blk, nh), lambda i: (i, 0)),
            pl.BlockSpec((blk, nh), lambda i: (i, 0)),
        ],
        out_shape=[
            jax.ShapeDtypeStruct((n, nh), jnp.bfloat16),
            jax.ShapeDtypeStruct((n, nh), jnp.bfloat16),
        ],
    )(seq, fc_W, fc_b.reshape(1, nh), w_top, w_bot, lin1_b.reshape(1, nh))


def _edge_scores(a_tab, b_tab, row3, col3, w2p, b2_vec):
    info = plsc.get_sparse_core_info()
    nc = info.num_cores
    nw = nc * info.num_subcores
    g_per_w = row3.shape[1]
    nh = 2 * a_tab.shape[1]
    ngrp = _CHUNK // _LANES
    nseg = nh // (2 * _LANES)
    mesh = plsc.VectorSubcoreMesh(core_axis_name="c", subcore_axis_name="s")

    @functools.partial(
        pl.kernel,
        out_type=jax.ShapeDtypeStruct((nw, g_per_w, _CHUNK), jnp.float32),
        mesh=mesh,
        compiler_params=pltpu.CompilerParams(
            needs_layout_passes=False, use_tc_tiling_on_sc=False
        ),
        scratch_types=[
            pltpu.VMEM((g_per_w, _CHUNK), jnp.int32),
            pltpu.VMEM((g_per_w, _CHUNK), jnp.int32),
        ]
        + [pltpu.VMEM((_CHUNK, nh // 2), jnp.int32) for _ in range(2 * _DEPTH)]
        + [
            pltpu.VMEM((g_per_w, _CHUNK), jnp.float32),
            pltpu.VMEM((nh,), jnp.float32),
            pltpu.VMEM((_LANES,), jnp.float32),
        ]
        + [pltpu.SemaphoreType.DMA for _ in range(2 * _DEPTH)],
    )
    def k(a_hbm, b_hbm, row_hbm, col_hbm, w2_hbm, b2_hbm, out_hbm, *refs):
        idxr_v, idxc_v = refs[0], refs[1]
        abufs = refs[2 : 2 + _DEPTH]
        bbufs = refs[2 + _DEPTH : 2 + 2 * _DEPTH]
        o_v, w2_v, b2_v = refs[2 + 2 * _DEPTH : 5 + 2 * _DEPTH]
        sems_a = refs[5 + 2 * _DEPTH : 5 + 3 * _DEPTH]
        sems_b = refs[5 + 3 * _DEPTH : 5 + 4 * _DEPTH]

        wid = lax.axis_index("s") * nc + lax.axis_index("c")
        pltpu.sync_copy(row_hbm.at[wid], idxr_v)
        pltpu.sync_copy(col_hbm.at[wid], idxc_v)
        pltpu.sync_copy(w2_hbm, w2_v)
        pltpu.sync_copy(b2_hbm, b2_v)
        lanes = lax.iota(jnp.int32, _LANES)
        b2 = b2_v[...]
        wseg = [w2_v[pl.ds(s * _LANES, _LANES)] for s in range(nh // _LANES)]

        def issue(g, p):
            pltpu.make_async_copy(a_hbm.at[idxr_v.at[g]], abufs[p], sems_a[p]).start()
            pltpu.make_async_copy(b_hbm.at[idxc_v.at[g]], bbufs[p], sems_b[p]).start()

        def drain(g, p):
            pltpu.make_async_copy(a_hbm.at[idxr_v.at[g]], abufs[p], sems_a[p]).wait()
            pltpu.make_async_copy(b_hbm.at[idxc_v.at[g]], bbufs[p], sems_b[p]).wait()

        def compute(g, p):
            a_buf, b_buf = abufs[p], bbufs[p]

            def egstep(eg, carry):
                r = jnp.zeros((_LANES,), jnp.float32)
                for j in range(_LANES):
                    e = eg * _LANES + j
                    acc = None
                    for s in range(nseg):
                        av = plsc.bitcast(
                            a_buf[e, pl.ds(s * _LANES, _LANES)], jnp.bfloat16
                        )
                        bv = plsc.bitcast(
                            b_buf[e, pl.ds(s * _LANES, _LANES)], jnp.bfloat16
                        )
                        h = jnp.maximum(av + bv, jnp.bfloat16(0.0))
                        pe, po = plsc.unpack(
                            h,
                            format=plsc.PackFormat.INTERLEAVED,
                            preferred_element_type=jnp.float32,
                        )
                        part = pe * wseg[2 * s] + po * wseg[2 * s + 1]
                        acc = part if acc is None else acc + part
                    total = jnp.sum(acc)
                    r = jnp.where(lanes == j, total, r)
                o_v[g, pl.ds(eg * _LANES, _LANES)] = r + b2
                return carry

            lax.fori_loop(0, ngrp, egstep, 0)

        for g0 in range(_DEPTH - 1):
            issue(g0, g0)

        def ring(gg, carry):
            for p in range(_DEPTH):
                g = _DEPTH * gg + p

                @pl.when(g + _DEPTH - 1 < g_per_w)
                def _():
                    issue(g + _DEPTH - 1, (p + _DEPTH - 1) % _DEPTH)

                drain(g, p)
                compute(g, p)
            return carry

        lax.fori_loop(0, g_per_w // _DEPTH, ring, 0)
        pltpu.sync_copy(o_v, out_hbm.at[wid])

    return k(a_tab, b_tab, row3, col3, w2p, b2_vec)


def kernel(seq, adj, fc_W, fc_b, lin1_W, lin1_b, lin2_W, lin2_b):
    e = adj.shape[1]
    nh = lin1_W.shape[0] // 2
    a_tab, b_tab = _node_tables(seq, fc_W, fc_b, lin1_W[:nh], lin1_W[nh:], lin1_b)
    n = a_tab.shape[0]
    a_tab = lax.bitcast_convert_type(a_tab.reshape(n, nh // 2, 2), jnp.int32)
    b_tab = lax.bitcast_convert_type(b_tab.reshape(n, nh // 2, 2), jnp.int32)

    info = plsc.get_sparse_core_info()
    nw = info.num_cores * info.num_subcores
    per = nw * _CHUNK
    g_per_w = -(-e // per)
    g_per_w += (-g_per_w) % _DEPTH
    e_pad = g_per_w * per
    row = adj[0].astype(jnp.int32)
    col = adj[1].astype(jnp.int32)
    if e_pad != e:
        zpad = jnp.zeros((e_pad - e,), jnp.int32)
        row = jnp.concatenate([row, zpad])
        col = jnp.concatenate([col, zpad])
    row3 = row.reshape(nw, g_per_w, _CHUNK)
    col3 = col.reshape(nw, g_per_w, _CHUNK)

    w2p = (
        lin2_W[:, 0]
        .reshape(nh // (2 * _LANES), _LANES, 2)
        .transpose(0, 2, 1)
        .reshape(nh)
    )
    b2_vec = jnp.full((_LANES,), lin2_b[0], jnp.float32)
    out = _edge_scores(a_tab, b_tab, row3, col3, w2p, b2_vec)
    return out.reshape(-1)[:e]
